# baseline (speedup 1.0000x reference)
import functools

import jax
import jax.numpy as jnp
from jax import lax
from jax.experimental import pallas as pl
from jax.experimental.pallas import tpu as pltpu
from jax.experimental.pallas import tpu_sc as plsc

N = 10000
E = 160000
D = 256
SL = 2
LN = 128
NC = 2
NS = 16
NHREAL = N // 2
NH = 5120
RPT = NH // NS
CAP = 163840
BM = 32
CAPB = CAP // BM
G = 8
SLOTS = 3

NPAD = 10240
EPAD = 163840
BD = 128
DB = EPAD // (NC * NS) // BD
DRPT = NPAD // NS

_mesh = plsc.VectorSubcoreMesh(
    core_axis_name="c", subcore_axis_name="s", num_cores=NC, num_subcores=NS
)


@functools.partial(
    pl.kernel,
    out_type=jax.ShapeDtypeStruct((NC, NPAD), jnp.float32),
    mesh=_mesh,
    scratch_types=[
        pltpu.VMEM((DB, BD), jnp.int32),
        pltpu.VMEM((BD,), jnp.float32),
        pltpu.VMEM((DRPT,), jnp.float32),
        pltpu.VMEM_SHARED((NPAD,), jnp.float32),
    ],
)
def _deg_kernel(dst_hbm, deg_hbm, dst_v, ones_v, zrow_v, deg_sh):
    c = lax.axis_index("c")
    s = lax.axis_index("s")
    pltpu.sync_copy(dst_hbm.at[c, s], dst_v)
    one16 = jnp.ones((16,), jnp.float32)
    z16 = jnp.zeros((16,), jnp.float32)

    def f1(i, _):
        ones_v[pl.ds(i * 16, 16)] = one16
        return 0

    lax.fori_loop(0, BD // 16, f1, 0)

    def f0(i, _):
        zrow_v[pl.ds(i * 16, 16)] = z16
        return 0

    lax.fori_loop(0, DRPT // 16, f0, 0)
    pltpu.sync_copy(zrow_v, deg_sh.at[pl.ds(s * DRPT, DRPT)])
    plsc.subcore_barrier()

    def body(j, _):
        pltpu.sync_copy(ones_v, deg_sh.at[dst_v.at[j]], add=True)
        return 0

    lax.fori_loop(0, DB, body, 0)
    plsc.subcore_barrier()
    pltpu.sync_copy(deg_sh.at[pl.ds(s * DRPT, DRPT)],
                    deg_hbm.at[c, pl.ds(s * DRPT, DRPT)])


@functools.partial(
    pl.kernel,
    out_type=jax.ShapeDtypeStruct((NC, NH, SL, LN), jnp.float32),
    mesh=_mesh,
    scratch_types=[
        pltpu.VMEM((SLOTS * G, BM), jnp.int32),
        pltpu.VMEM((SLOTS * G, BM), jnp.int32),
        pltpu.VMEM((4, BM, SL, LN), jnp.float32),
        pltpu.VMEM_SHARED((NH, SL, LN), jnp.float32),
        pltpu.VMEM((16,), jnp.int32),
        pltpu.SemaphoreType.DMA,
        pltpu.SemaphoreType.DMA,
        pltpu.SemaphoreType.DMA,
        pltpu.SemaphoreType.DMA,
        pltpu.SemaphoreType.DMA,
        pltpu.SemaphoreType.DMA,
        pltpu.SemaphoreType.DMA,
        pltpu.SemaphoreType.DMA,
    ],
)
def _scatter_kernel(y_hbm, srcs_hbm, dstl_hbm, tbt_hbm, out_hbm,
                    src_v, dst_v, bufs, acc_sh, cnt_sm,
                    sg0, sg1, sg2, sg3, ss0, ss1, ss2, ss3):
    c = lax.axis_index("c")
    s = lax.axis_index("s")
    sg = [sg0, sg1, sg2, sg3]
    ss = [ss0, ss1, ss2, ss3]
    pltpu.sync_copy(tbt_hbm, cnt_sm)
    cnt16 = cnt_sm[...]
    tbt = jnp.where(c == 0, cnt16[0], cnt16[1])
    tbt = pl.multiple_of(tbt, 8)
    base = s * tbt
    ng = (tbt + G - 1) // G

    z16 = jnp.zeros((16,), jnp.float32)

    def zrow(i, _):
        def zsub(q, __):
            def zcol(jc, ___):
                bufs[0, i, q, pl.ds(jc * 16, 16)] = z16
                return 0
            return lax.fori_loop(0, LN // 16, zcol, 0)
        return lax.fori_loop(0, SL, zsub, 0)

    lax.fori_loop(0, BM, zrow, 0)

    def zcp(k, _):
        pltpu.sync_copy(bufs.at[0], acc_sh.at[pl.ds(s * RPT + k * BM, BM)])
        return 0

    lax.fori_loop(0, RPT // BM, zcp, 0)
    plsc.subcore_barrier()

    def idxrow(b):
        return ((b // G) % SLOTS) * G + b % G

    def load_idx_group(g):
        row = (g % SLOTS) * G
        pltpu.sync_copy(srcs_hbm.at[c, pl.ds(base + g * G, G)],
                        src_v.at[pl.ds(row, G)])
        pltpu.sync_copy(dstl_hbm.at[c, pl.ds(base + g * G, G)],
                        dst_v.at[pl.ds(row, G)])

    def gather(b, m, sem):
        pltpu.async_copy(y_hbm.at[src_v.at[idxrow(b)]], bufs.at[m], sem)

    def wait_gather(m, sem):
        pltpu.make_async_copy(y_hbm.at[src_v.at[0]], bufs.at[m], sem).wait()

    def scatter(b, m, sem):
        pltpu.async_copy(bufs.at[m], acc_sh.at[dst_v.at[idxrow(b)]], sem,
                         add=True)

    def wait_scatter(m, sem):
        pltpu.make_async_copy(bufs.at[m], acc_sh.at[dst_v.at[0]], sem).wait()

    load_idx_group(0)
    gather(0, 0, sg[0])
    gather(1, 1, sg[1])
    gather(2, 2, sg[2])

    def body(j, _):
        for m in range(4):
            @pl.when((j + 3) % 4 == m)
            def _(m=m):
                @pl.when(j >= 1)
                def _():
                    wait_scatter(m, ss[m])

                @pl.when(jnp.logical_and((j + 3) % G == 0,
                                         (j + 3) // G < ng))
                def _():
                    load_idx_group((j + 3) // G)

                @pl.when(j + 3 < tbt)
                def _():
                    gather(j + 3, m, sg[m])

        for k in range(4):
            @pl.when(j % 4 == k)
            def _(k=k):
                wait_gather(k, sg[k])
                scatter(j, k, ss[k])
        return 0

    lax.fori_loop(0, tbt, body, 0)
    for m in range(4):
        @pl.when((tbt - 1) % 4 == m)
        def _(m=m):
            wait_scatter(m, ss[m])
    plsc.subcore_barrier()
    pltpu.sync_copy(acc_sh.at[pl.ds(s * RPT, RPT)],
                    out_hbm.at[c, pl.ds(s * RPT, RPT)])


RA = 1000
RB = 1000


def _mm_scale_body(x_ref, w_ref, dinv_ref, y_ref):
    z = jnp.dot(x_ref[...], w_ref[...], preferred_element_type=jnp.float32)
    y_ref[...] = dinv_ref[...] * z


def _layer2_body(acc_ref, y1_ref, dinv_ref, b1_ref, w2_ref, y2_ref):
    h = jnp.tanh(dinv_ref[...] * (acc_ref[0] + y1_ref[...]) + b1_ref[...])
    z2 = jnp.dot(h, w2_ref[...], preferred_element_type=jnp.float32)
    y2_ref[...] = dinv_ref[...] * z2


def _final_body(acc_ref, y2_ref, dinv_ref, b2_ref, out_ref):
    out_ref[...] = dinv_ref[...] * (acc_ref[0] + y2_ref[...]) + b2_ref[...]


_HB = NHREAL // RB


def kernel(latent, edge_index, W1, b1, W2, b2):
    src = edge_index[0].astype(jnp.int32)
    dst = edge_index[1].astype(jnp.int32)

    pad = EPAD - E
    dstp = jnp.concatenate([dst, jnp.full((pad,), N, jnp.int32)])
    dst_deg = dstp.reshape(NC, NS, DB, BD)

    bit = (dst >= NHREAL).astype(jnp.int32)
    c1 = jnp.cumsum(bit)
    c0 = jnp.arange(1, E + 1, dtype=jnp.int32) - c1
    n0 = c0[-1]
    n1 = E - n0
    pos = jnp.where(bit == 0, c0 - 1, CAP + c1 - 1)
    srcs_p = jnp.zeros((2 * CAP,), jnp.int32).at[pos].set(src)
    dstl_p = jnp.full((2 * CAP,), NHREAL, jnp.int32).at[pos].set(
        dst - bit * NHREAL)
    srcs_p = srcs_p.reshape(NC, CAPB, BM)
    dstl_p = dstl_p.reshape(NC, CAPB, BM)
    per_tile = BM * NS
    tbt = (jnp.stack([n0, n1]) + per_tile - 1) // per_tile
    tbt = (tbt + 7) // 8 * 8
    tbt = jnp.maximum(tbt, 8)
    tbt_arr = jnp.zeros((16,), jnp.int32).at[0:2].set(tbt)

    deg2 = _deg_kernel(dst_deg)
    dinv = lax.rsqrt(deg2[0, :N] + deg2[1, :N] + 1.0)[:, None]

    y1 = pl.pallas_call(
        _mm_scale_body,
        grid=(N // RA,),
        in_specs=[
            pl.BlockSpec((RA, D), lambda i: (i, 0)),
            pl.BlockSpec((D, D), lambda i: (0, 0)),
            pl.BlockSpec((RA, 1), lambda i: (i, 0)),
        ],
        out_specs=pl.BlockSpec((RA, D), lambda i: (i, 0)),
        out_shape=jax.ShapeDtypeStruct((N, D), jnp.float32),
    )(latent, W1, dinv)

    acc1 = _scatter_kernel(y1.reshape(N, SL, LN), srcs_p, dstl_p, tbt_arr)
    acc1 = acc1.reshape(NC, NH, D)

    acc_spec = pl.BlockSpec((1, RB, D), lambda i: (i // _HB, i % _HB, 0))
    y2 = pl.pallas_call(
        _layer2_body,
        grid=(N // RB,),
        in_specs=[
            acc_spec,
            pl.BlockSpec((RB, D), lambda i: (i, 0)),
            pl.BlockSpec((RB, 1), lambda i: (i, 0)),
            pl.BlockSpec((1, D), lambda i: (0, 0)),
            pl.BlockSpec((D, D), lambda i: (0, 0)),
        ],
        out_specs=pl.BlockSpec((RB, D), lambda i: (i, 0)),
        out_shape=jax.ShapeDtypeStruct((N, D), jnp.float32),
    )(acc1, y1, dinv, b1.reshape(1, D), W2)

    acc2 = _scatter_kernel(y2.reshape(N, SL, LN), srcs_p, dstl_p, tbt_arr)
    acc2 = acc2.reshape(NC, NH, D)

    out = pl.pallas_call(
        _final_body,
        grid=(N // RB,),
        in_specs=[
            acc_spec,
            pl.BlockSpec((RB, D), lambda i: (i, 0)),
            pl.BlockSpec((RB, 1), lambda i: (i, 0)),
            pl.BlockSpec((1, D), lambda i: (0, 0)),
        ],
        out_specs=pl.BlockSpec((RB, D), lambda i: (i, 0)),
        out_shape=jax.ShapeDtypeStruct((N, D), jnp.float32),
    )(acc2, y2, dinv, b2.reshape(1, D))

    return out


# speedup vs baseline: 2.5152x; 2.5152x over previous
import functools

import jax
import jax.numpy as jnp
from jax import lax
from jax.experimental import pallas as pl
from jax.experimental.pallas import tpu as pltpu
from jax.experimental.pallas import tpu_sc as plsc

N = 10000
E = 160000
D = 256
HALF = 128
NC = 2
NS = 16
NPAD = 10240
EPAD = 163840
BD = 128
DB = EPAD // (NC * NS) // BD
BM = 80
MB = EPAD // NS // BM
G = 8
NG = MB // G
SLOTS = 3
RPT = NPAD // NS

_mesh = plsc.VectorSubcoreMesh(
    core_axis_name="c", subcore_axis_name="s", num_cores=NC, num_subcores=NS
)


def _fill_zero_2d(ref, rows):
    z16 = jnp.zeros((16,), jnp.float32)

    def row(i, _):
        def col(j, __):
            ref[i, pl.ds(j * 16, 16)] = z16
            return 0
        return lax.fori_loop(0, HALF // 16, col, 0)

    lax.fori_loop(0, rows, row, 0)


@functools.partial(
    pl.kernel,
    out_type=jax.ShapeDtypeStruct((NC, NPAD), jnp.float32),
    mesh=_mesh,
    scratch_types=[
        pltpu.VMEM((DB, BD), jnp.int32),
        pltpu.VMEM((BD,), jnp.float32),
        pltpu.VMEM((RPT,), jnp.float32),
        pltpu.VMEM_SHARED((NPAD,), jnp.float32),
    ],
)
def _deg_kernel(dst_hbm, deg_hbm, dst_v, ones_v, zrow_v, deg_sh):
    c = lax.axis_index("c")
    s = lax.axis_index("s")
    pltpu.sync_copy(dst_hbm.at[c, s], dst_v)
    one16 = jnp.ones((16,), jnp.float32)
    z16 = jnp.zeros((16,), jnp.float32)

    def f1(i, _):
        ones_v[pl.ds(i * 16, 16)] = one16
        return 0

    lax.fori_loop(0, BD // 16, f1, 0)

    def f0(i, _):
        zrow_v[pl.ds(i * 16, 16)] = z16
        return 0

    lax.fori_loop(0, RPT // 16, f0, 0)
    pltpu.sync_copy(zrow_v, deg_sh.at[pl.ds(s * RPT, RPT)])
    plsc.subcore_barrier()

    def body(j, _):
        pltpu.sync_copy(ones_v, deg_sh.at[dst_v.at[j]], add=True)
        return 0

    lax.fori_loop(0, DB, body, 0)
    plsc.subcore_barrier()
    pltpu.sync_copy(deg_sh.at[pl.ds(s * RPT, RPT)], deg_hbm.at[c, pl.ds(s * RPT, RPT)])


@functools.partial(
    pl.kernel,
    out_type=jax.ShapeDtypeStruct((NC, NPAD, HALF), jnp.float32),
    mesh=_mesh,
    scratch_types=[
        pltpu.VMEM((SLOTS * G, BM), jnp.int32),
        pltpu.VMEM((SLOTS * G, BM), jnp.int32),
        pltpu.VMEM((4, BM, HALF), jnp.float32),
        pltpu.VMEM_SHARED((NPAD, HALF), jnp.float32),
        pltpu.SemaphoreType.DMA,
        pltpu.SemaphoreType.DMA,
        pltpu.SemaphoreType.DMA,
        pltpu.SemaphoreType.DMA,
        pltpu.SemaphoreType.DMA,
        pltpu.SemaphoreType.DMA,
        pltpu.SemaphoreType.DMA,
        pltpu.SemaphoreType.DMA,
    ],
)
def _scatter_kernel(y_hbm, src2_hbm, dst_hbm, out_hbm,
                    src2_v, dst_v, bufs, acc_sh,
                    sg0, sg1, sg2, sg3, ss0, ss1, ss2, ss3):
    c = lax.axis_index("c")
    s = lax.axis_index("s")
    sg = [sg0, sg1, sg2, sg3]
    ss = [ss0, ss1, ss2, ss3]
    z16 = jnp.zeros((16,), jnp.float32)

    def zrow(i, _):
        def zcol(jc, __):
            bufs[0, i, pl.ds(jc * 16, 16)] = z16
            return 0
        return lax.fori_loop(0, HALF // 16, zcol, 0)

    lax.fori_loop(0, BM, zrow, 0)

    def zcp(k, _):
        pltpu.sync_copy(bufs.at[0], acc_sh.at[pl.ds(s * RPT + k * BM, BM)])
        return 0

    lax.fori_loop(0, RPT // BM, zcp, 0)
    plsc.subcore_barrier()

    def idxrow(b):
        return ((b // G) % SLOTS) * G + b % G

    def load_idx_group(g):
        row = (g % SLOTS) * G
        pltpu.sync_copy(src2_hbm.at[c, s, pl.ds(g * G, G)],
                        src2_v.at[pl.ds(row, G)])
        pltpu.sync_copy(dst_hbm.at[s, pl.ds(g * G, G)],
                        dst_v.at[pl.ds(row, G)])

    def gather(b, m, sem):
        pltpu.async_copy(y_hbm.at[src2_v.at[idxrow(b)]], bufs.at[m], sem)

    def wait_gather(m, sem):
        pltpu.make_async_copy(y_hbm.at[src2_v.at[0]], bufs.at[m], sem).wait()

    def scatter(b, m, sem):
        pltpu.async_copy(bufs.at[m], acc_sh.at[dst_v.at[idxrow(b)]], sem,
                         add=True)

    def wait_scatter(m, sem):
        pltpu.make_async_copy(bufs.at[m], acc_sh.at[dst_v.at[0]], sem).wait()

    load_idx_group(0)
    gather(0, 0, sg[0])
    gather(1, 1, sg[1])
    gather(2, 2, sg[2])

    def body(j, _):
        for m in range(4):
            @pl.when((j + 3) % 4 == m)
            def _(m=m):
                @pl.when(j >= 1)
                def _():
                    wait_scatter(m, ss[m])

                @pl.when(jnp.logical_and((j + 3) % G == 0,
                                         (j + 3) // G < NG))
                def _():
                    load_idx_group((j + 3) // G)

                @pl.when(j + 3 < MB)
                def _():
                    gather(j + 3, m, sg[m])

        for k in range(4):
            @pl.when(j % 4 == k)
            def _(k=k):
                wait_gather(k, sg[k])
                scatter(j, k, ss[k])
        return 0

    lax.fori_loop(0, MB, body, 0)
    wait_scatter((MB - 1) % 4, ss[(MB - 1) % 4])
    plsc.subcore_barrier()
    pltpu.sync_copy(acc_sh.at[pl.ds(s * RPT, RPT)], out_hbm.at[c, pl.ds(s * RPT, RPT)])


R = 1000


def _mm_scale_body(x_ref, w_ref, dinv_ref, y_ref):
    z = jnp.dot(x_ref[...], w_ref[...], preferred_element_type=jnp.float32)
    y_ref[...] = dinv_ref[...] * z


def _layer2_body(acc_ref, y1_ref, dinv_ref, b1_ref, w2_ref, y2_ref):
    sagg = jnp.concatenate([acc_ref[0], acc_ref[1]], axis=1)
    h = jnp.tanh(dinv_ref[...] * (sagg + y1_ref[...]) + b1_ref[...])
    z2 = jnp.dot(h, w2_ref[...], preferred_element_type=jnp.float32)
    y2_ref[...] = dinv_ref[...] * z2


def _final_body(acc_ref, y2_ref, dinv_ref, b2_ref, out_ref):
    sagg = jnp.concatenate([acc_ref[0], acc_ref[1]], axis=1)
    out_ref[...] = dinv_ref[...] * (sagg + y2_ref[...]) + b2_ref[...]


def kernel(latent, edge_index, W1, b1, W2, b2):
    src = edge_index[0].astype(jnp.int32)
    dst = edge_index[1].astype(jnp.int32)
    pad = EPAD - E
    srcp = jnp.concatenate([src, jnp.zeros((pad,), jnp.int32)])
    dstp = jnp.concatenate([dst, jnp.full((pad,), N, jnp.int32)])
    src2 = jnp.stack([srcp * 2, srcp * 2 + 1]).reshape(NC, NS, MB, BM)
    dst_main = dstp.reshape(NS, MB, BM)
    dst_deg = dstp.reshape(NC, NS, DB, BD)

    deg2 = _deg_kernel(dst_deg)
    dinv = lax.rsqrt(deg2[0, :N] + deg2[1, :N] + 1.0)[:, None]

    grid = N // R
    y1 = pl.pallas_call(
        _mm_scale_body,
        grid=(grid,),
        in_specs=[
            pl.BlockSpec((R, D), lambda i: (i, 0)),
            pl.BlockSpec((D, D), lambda i: (0, 0)),
            pl.BlockSpec((R, 1), lambda i: (i, 0)),
        ],
        out_specs=pl.BlockSpec((R, D), lambda i: (i, 0)),
        out_shape=jax.ShapeDtypeStruct((N, D), jnp.float32),
    )(latent, W1, dinv)

    acc1 = _scatter_kernel(y1.reshape(2 * N, HALF), src2, dst_main)

    y2 = pl.pallas_call(
        _layer2_body,
        grid=(grid,),
        in_specs=[
            pl.BlockSpec((NC, R, HALF), lambda i: (0, i, 0)),
            pl.BlockSpec((R, D), lambda i: (i, 0)),
            pl.BlockSpec((R, 1), lambda i: (i, 0)),
            pl.BlockSpec((1, D), lambda i: (0, 0)),
            pl.BlockSpec((D, D), lambda i: (0, 0)),
        ],
        out_specs=pl.BlockSpec((R, D), lambda i: (i, 0)),
        out_shape=jax.ShapeDtypeStruct((N, D), jnp.float32),
    )(acc1, y1, dinv, b1.reshape(1, D), W2)

    acc2 = _scatter_kernel(y2.reshape(2 * N, HALF), src2, dst_main)

    out = pl.pallas_call(
        _final_body,
        grid=(grid,),
        in_specs=[
            pl.BlockSpec((NC, R, HALF), lambda i: (0, i, 0)),
            pl.BlockSpec((R, D), lambda i: (i, 0)),
            pl.BlockSpec((R, 1), lambda i: (i, 0)),
            pl.BlockSpec((1, D), lambda i: (0, 0)),
        ],
        out_specs=pl.BlockSpec((R, D), lambda i: (i, 0)),
        out_shape=jax.ShapeDtypeStruct((N, D), jnp.float32),
    )(acc2, y2, dinv, b2.reshape(1, D))

    return out
